# baseline (speedup 1.0000x reference)
import functools

import jax
import jax.numpy as jnp
from jax import lax
from jax.experimental import pallas as pl
from jax.experimental.pallas import tpu as pltpu
from jax.experimental.pallas import tpu_sc as plsc

B, C, H, W = 4, 96, 224, 224
PLANE = H * W
NPLANES = B * C
NC, NS = 2, 16
NWORKERS = NC * NS
PLANES_PER_W = NPLANES // NWORKERS
L = 16

CHUNK_ROWS = 56
CHUNK = CHUNK_ROWS * W
NCHUNKS = H // CHUNK_ROWS
VECS_PER_ROW = W // L


def _advect_body(img_hbm, u_hbm, v_hbm, out_hbm, bufs, acc, sems):
    wid = lax.axis_index("s") * NC + lax.axis_index("c")
    first_plane = wid * PLANES_PER_W

    lane = lax.iota(jnp.int32, L).astype(jnp.float32)
    zeros = jnp.zeros((L,), jnp.float32)
    wmax = jnp.float32(W - 1)
    hmax = jnp.float32(H - 1)
    fzero = jnp.float32(0.0)
    fone = jnp.float32(1.0)

    def start_chunk(plane, ch, slot):
        off = plane * PLANE + ch * CHUNK
        for a, (hbm, _) in enumerate(((img_hbm, 0), (u_hbm, 1), (v_hbm, 2))):
            pltpu.async_copy(hbm.at[pl.ds(off, CHUNK)], bufs[slot][a], sems[slot])

    def wait_chunk(slot):
        for a in range(3):
            pltpu.make_async_copy(img_hbm.at[pl.ds(0, CHUNK)], bufs[slot][a],
                                  sems[slot]).wait()

    start_chunk(first_plane, 0, 0)

    def per_plane(p, carry):
        plane = first_plane + p

        for ch in range(NCHUNKS):
            slot = ch % 2
            is_last = (p == PLANES_PER_W - 1) & (ch == NCHUNKS - 1)

            @pl.when(jnp.logical_not(is_last))
            def _():
                nch = (ch + 1) % NCHUNKS
                nplane = plane + (1 if ch == NCHUNKS - 1 else 0)
                start_chunk(nplane, nch, 1 - slot)

            wait_chunk(slot)
            img_v, u_v, v_v = bufs[slot]

            def per_row(r, _):
                ybase = (ch * CHUNK_ROWS + r).astype(jnp.float32)
                for j in range(VECS_PER_ROW):
                    sl = pl.ds(r * W + j * L, L)
                    img = img_v[sl]
                    X = jnp.minimum(jnp.maximum(lane + jnp.float32(j * L)
                                                + u_v[sl], fzero), wmax)
                    Y = jnp.minimum(jnp.maximum(ybase + v_v[sl], fzero), hmax)
                    x0 = X.astype(jnp.int32)
                    y0 = Y.astype(jnp.int32)
                    x0f = x0.astype(jnp.float32)
                    y0f = y0.astype(jnp.float32)
                    x1f = jnp.minimum(x0f + fone, wmax)
                    y1f = jnp.minimum(y0f + fone, hmax)
                    x1 = x1f.astype(jnp.int32)
                    y1 = y1f.astype(jnp.int32)
                    dx = X - x0f
                    dy = Y - y0f
                    gx = x1f - X
                    gy = y1f - Y
                    r0 = y0 * W
                    r1 = y1 * W
                    vdx = img * dx
                    vgx = img * gx
                    plsc.addupdate_scatter(acc, [r0 + x0], vdx * dy)
                    plsc.addupdate_scatter(acc, [r1 + x0], vdx * gy)
                    plsc.addupdate_scatter(acc, [r0 + x1], vgx * dy)
                    plsc.addupdate_scatter(acc, [r1 + x1], vgx * gy)
                return _
            lax.fori_loop(0, CHUNK_ROWS, per_row, 0)

        pltpu.sync_copy(acc, out_hbm.at[pl.ds(plane * PLANE, PLANE)])

        def zero_body(i, _):
            acc[pl.ds(i * L, L)] = zeros
            return _
        lax.fori_loop(0, PLANE // L, zero_body, 0, unroll=4)
        return carry

    def zero_body0(i, _):
        acc[pl.ds(i * L, L)] = zeros
        return _
    lax.fori_loop(0, PLANE // L, zero_body0, 0, unroll=4)

    lax.fori_loop(0, PLANES_PER_W, per_plane, 0)


@jax.jit
def kernel(input_image, U, V):
    mesh = plsc.VectorSubcoreMesh(core_axis_name="c", subcore_axis_name="s",
                                  num_cores=NC, num_subcores=NS)
    run = pl.kernel(
        _advect_body,
        out_type=jax.ShapeDtypeStruct((NPLANES * PLANE,), jnp.float32),
        mesh=mesh,
        scratch_types=[
            [[pltpu.VMEM((CHUNK,), jnp.float32) for _ in range(3)]
             for _ in range(2)],
            pltpu.VMEM((PLANE,), jnp.float32),
            [pltpu.SemaphoreType.DMA for _ in range(2)],
        ],
        compiler_params=pltpu.CompilerParams(needs_layout_passes=False),
    )
    out = run(input_image.reshape(-1), U.reshape(-1), V.reshape(-1))
    return out.reshape(B, C, H, W)


# speedup vs baseline: 1.5457x; 1.5457x over previous
import functools

import jax
import jax.numpy as jnp
from jax import lax
from jax.experimental import pallas as pl
from jax.experimental.pallas import tpu as pltpu
from jax.experimental.pallas import tpu_sc as plsc

B, C, H, W = 4, 96, 224, 224
PLANE = H * W
NPLANES = B * C
NC, NS = 2, 16
NWORKERS = NC * NS
PLANES_PER_W = NPLANES // NWORKERS
L = 16

CHUNK_ROWS = 56
CHUNK = CHUNK_ROWS * W
NCHUNKS = H // CHUNK_ROWS
VECS_PER_ROW = W // L
GROUP = 4


def _advect_body(img_hbm, u_hbm, v_hbm, out_hbm, bufs, acc, sems):
    wid = lax.axis_index("s") * NC + lax.axis_index("c")
    first_plane = wid * PLANES_PER_W

    lane = lax.iota(jnp.int32, L).astype(jnp.float32)
    zeros = jnp.zeros((L,), jnp.float32)
    wmax = jnp.float32(W - 1)
    hmax = jnp.float32(H - 1)
    fzero = jnp.float32(0.0)
    fone = jnp.float32(1.0)

    def start_chunk(plane, ch, slot):
        off = plane * PLANE + ch * CHUNK
        for a, (hbm, _) in enumerate(((img_hbm, 0), (u_hbm, 1), (v_hbm, 2))):
            pltpu.async_copy(hbm.at[pl.ds(off, CHUNK)], bufs[slot][a], sems[slot])

    def wait_chunk(slot):
        for a in range(3):
            pltpu.make_async_copy(img_hbm.at[pl.ds(0, CHUNK)], bufs[slot][a],
                                  sems[slot]).wait()

    start_chunk(first_plane, 0, 0)

    def per_plane(p, carry):
        plane = first_plane + p

        for ch in range(NCHUNKS):
            slot = ch % 2
            is_last = (p == PLANES_PER_W - 1) & (ch == NCHUNKS - 1)

            @pl.when(jnp.logical_not(is_last))
            def _():
                nch = (ch + 1) % NCHUNKS
                nplane = plane + (1 if ch == NCHUNKS - 1 else 0)
                start_chunk(nplane, nch, 1 - slot)

            wait_chunk(slot)
            img_v, u_v, v_v = bufs[slot]

            def per_row(r, rcarry):
                ybase = (ch * CHUNK_ROWS + r).astype(jnp.float32)

                def load_group(g):
                    js = range(g * GROUP, min((g + 1) * GROUP, VECS_PER_ROW))
                    sls = [pl.ds(r * W + j * L, L) for j in js]
                    return ([img_v[sl] for sl in sls],
                            [u_v[sl] for sl in sls],
                            [v_v[sl] for sl in sls])

                def compute_group(g, data):
                    imgs, us, vs = data
                    js = range(g * GROUP, min((g + 1) * GROUP, VECS_PER_ROW))
                    for k, j in enumerate(js):
                        img = imgs[k]
                        X = jnp.minimum(jnp.maximum(lane + jnp.float32(j * L)
                                                    + us[k], fzero), wmax)
                        Y = jnp.minimum(jnp.maximum(ybase + vs[k], fzero), hmax)
                        x0 = X.astype(jnp.int32)
                        y0 = Y.astype(jnp.int32)
                        x0f = x0.astype(jnp.float32)
                        y0f = y0.astype(jnp.float32)
                        x1f = jnp.minimum(x0f + fone, wmax)
                        y1f = jnp.minimum(y0f + fone, hmax)
                        x1 = x1f.astype(jnp.int32)
                        y1 = y1f.astype(jnp.int32)
                        dx = X - x0f
                        dy = Y - y0f
                        gx = x1f - X
                        gy = y1f - Y
                        r0 = y0 * W
                        r1 = y1 * W
                        vdx = img * dx
                        vgx = img * gx
                        plsc.addupdate_scatter(acc, [r0 + x0], vdx * dy)
                        plsc.addupdate_scatter(acc, [r1 + x0], vdx * gy)
                        plsc.addupdate_scatter(acc, [r0 + x1], vgx * dy)
                        plsc.addupdate_scatter(acc, [r1 + x1], vgx * gy)

                ngroups = -(-VECS_PER_ROW // GROUP)
                data = load_group(0)
                for g in range(ngroups):
                    nxt = load_group(g + 1) if g + 1 < ngroups else None
                    compute_group(g, data)
                    data = nxt
                return rcarry
            lax.fori_loop(0, CHUNK_ROWS, per_row, 0)

        pltpu.sync_copy(acc, out_hbm.at[pl.ds(plane * PLANE, PLANE)])

        def zero_body(i, _):
            acc[pl.ds(i * L, L)] = zeros
            return _
        lax.fori_loop(0, PLANE // L, zero_body, 0, unroll=4)
        return carry

    def zero_body0(i, _):
        acc[pl.ds(i * L, L)] = zeros
        return _
    lax.fori_loop(0, PLANE // L, zero_body0, 0, unroll=4)

    lax.fori_loop(0, PLANES_PER_W, per_plane, 0)


@jax.jit
def kernel(input_image, U, V):
    mesh = plsc.VectorSubcoreMesh(core_axis_name="c", subcore_axis_name="s",
                                  num_cores=NC, num_subcores=NS)
    run = pl.kernel(
        _advect_body,
        out_type=jax.ShapeDtypeStruct((NPLANES * PLANE,), jnp.float32),
        mesh=mesh,
        scratch_types=[
            [[pltpu.VMEM((CHUNK,), jnp.float32) for _ in range(3)]
             for _ in range(2)],
            pltpu.VMEM((PLANE,), jnp.float32),
            [pltpu.SemaphoreType.DMA for _ in range(2)],
        ],
        compiler_params=pltpu.CompilerParams(needs_layout_passes=False),
    )
    out = run(input_image.reshape(-1), U.reshape(-1), V.reshape(-1))
    return out.reshape(B, C, H, W)


# speedup vs baseline: 1.6333x; 1.0567x over previous
import functools

import jax
import jax.numpy as jnp
from jax import lax
from jax.experimental import pallas as pl
from jax.experimental.pallas import tpu as pltpu
from jax.experimental.pallas import tpu_sc as plsc

B, C, H, W = 4, 96, 224, 224
PLANE = H * W
NPLANES = C
NC, NS = 2, 16
NWORKERS = NC * NS
PLANES_PER_W = NPLANES // NWORKERS
L = 16

CHUNK_ROWS = 56
CHUNK = CHUNK_ROWS * W
NCHUNKS = H // CHUNK_ROWS
VECS_PER_ROW = W // L
GROUP = 4


def _advect_body(img_hbm, u_hbm, v_hbm, out_hbm, bufs, acc, sems):
    wid = lax.axis_index("s") * NC + lax.axis_index("c")
    first_plane = wid * PLANES_PER_W

    lane = lax.iota(jnp.int32, L).astype(jnp.float32)
    zeros = jnp.zeros((L,), jnp.float32)
    wmax = jnp.float32(W - 1)
    hmax = jnp.float32(H - 1)
    fzero = jnp.float32(0.0)
    fone = jnp.float32(1.0)

    def start_chunk(plane, ch, slot):
        off = plane * PLANE + ch * CHUNK
        for a, (hbm, _) in enumerate(((img_hbm, 0), (u_hbm, 1), (v_hbm, 2))):
            pltpu.async_copy(hbm.at[pl.ds(off, CHUNK)], bufs[slot][a], sems[slot])

    def wait_chunk(slot):
        for a in range(3):
            pltpu.make_async_copy(img_hbm.at[pl.ds(0, CHUNK)], bufs[slot][a],
                                  sems[slot]).wait()

    start_chunk(first_plane, 0, 0)

    def per_plane(p, carry):
        plane = first_plane + p

        for ch in range(NCHUNKS):
            slot = ch % 2
            is_last = (p == PLANES_PER_W - 1) & (ch == NCHUNKS - 1)

            @pl.when(jnp.logical_not(is_last))
            def _():
                nch = (ch + 1) % NCHUNKS
                nplane = plane + (1 if ch == NCHUNKS - 1 else 0)
                start_chunk(nplane, nch, 1 - slot)

            wait_chunk(slot)
            img_v, u_v, v_v = bufs[slot]

            def per_row(r, rcarry):
                ybase = (ch * CHUNK_ROWS + r).astype(jnp.float32)

                def load_group(g):
                    js = range(g * GROUP, min((g + 1) * GROUP, VECS_PER_ROW))
                    sls = [pl.ds(r * W + j * L, L) for j in js]
                    return ([img_v[sl] for sl in sls],
                            [u_v[sl] for sl in sls],
                            [v_v[sl] for sl in sls])

                def compute_group(g, data):
                    imgs, us, vs = data
                    js = range(g * GROUP, min((g + 1) * GROUP, VECS_PER_ROW))
                    for k, j in enumerate(js):
                        img = imgs[k]
                        X = jnp.minimum(jnp.maximum(lane + jnp.float32(j * L)
                                                    + us[k], fzero), wmax)
                        Y = jnp.minimum(jnp.maximum(ybase + vs[k], fzero), hmax)
                        x0 = X.astype(jnp.int32)
                        y0 = Y.astype(jnp.int32)
                        x0f = x0.astype(jnp.float32)
                        y0f = y0.astype(jnp.float32)
                        x1f = jnp.minimum(x0f + fone, wmax)
                        y1f = jnp.minimum(y0f + fone, hmax)
                        x1 = x1f.astype(jnp.int32)
                        y1 = y1f.astype(jnp.int32)
                        dx = X - x0f
                        dy = Y - y0f
                        gx = x1f - X
                        gy = y1f - Y
                        r0 = y0 * W
                        r1 = y1 * W
                        vdx = img * dx
                        vgx = img * gx
                        plsc.addupdate_scatter(acc, [r0 + x0], vdx * dy)
                        plsc.addupdate_scatter(acc, [r1 + x0], vdx * gy)
                        plsc.addupdate_scatter(acc, [r0 + x1], vgx * dy)
                        plsc.addupdate_scatter(acc, [r1 + x1], vgx * gy)

                ngroups = -(-VECS_PER_ROW // GROUP)
                data = load_group(0)
                for g in range(ngroups):
                    nxt = load_group(g + 1) if g + 1 < ngroups else None
                    compute_group(g, data)
                    data = nxt
                return rcarry
            lax.fori_loop(0, CHUNK_ROWS, per_row, 0)

        pltpu.sync_copy(acc, out_hbm.at[pl.ds(plane * PLANE, PLANE)])

        def zero_body(i, _):
            acc[pl.ds(i * L, L)] = zeros
            return _
        lax.fori_loop(0, PLANE // L, zero_body, 0, unroll=4)
        return carry

    def zero_body0(i, _):
        acc[pl.ds(i * L, L)] = zeros
        return _
    lax.fori_loop(0, PLANE // L, zero_body0, 0, unroll=4)

    lax.fori_loop(0, PLANES_PER_W, per_plane, 0)


@jax.jit
def kernel(input_image, U, V):
    mesh = plsc.VectorSubcoreMesh(core_axis_name="c", subcore_axis_name="s",
                                  num_cores=NC, num_subcores=NS)
    run = pl.kernel(
        _advect_body,
        out_type=jax.ShapeDtypeStruct((NPLANES * PLANE,), jnp.float32),
        mesh=mesh,
        scratch_types=[
            [[pltpu.VMEM((CHUNK,), jnp.float32) for _ in range(3)]
             for _ in range(2)],
            pltpu.VMEM((PLANE,), jnp.float32),
            [pltpu.SemaphoreType.DMA for _ in range(2)],
        ],
        compiler_params=pltpu.CompilerParams(needs_layout_passes=False),
    )
    outs = [
        run(input_image[k].reshape(-1), U[k].reshape(-1), V[k].reshape(-1))
        for k in range(B)
    ]
    return jnp.stack([o.reshape(C, H, W) for o in outs])


# speedup vs baseline: 1.7202x; 1.0532x over previous
import functools

import jax
import jax.numpy as jnp
from jax import lax
from jax.experimental import pallas as pl
from jax.experimental.pallas import tpu as pltpu
from jax.experimental.pallas import tpu_sc as plsc

B, C, H, W = 4, 96, 224, 224
PLANE = H * W
NC, NS = 2, 16
NWORKERS = NC * NS
NPLANES = NWORKERS
NGROUPS = (B * C) // NPLANES
PLANES_PER_W = NPLANES // NWORKERS
L = 16

CHUNK_ROWS = 56
CHUNK = CHUNK_ROWS * W
NCHUNKS = H // CHUNK_ROWS
VECS_PER_ROW = W // L
GROUP = 4


def _advect_body(img_hbm, u_hbm, v_hbm, out_hbm, bufs, acc, sems):
    wid = lax.axis_index("s") * NC + lax.axis_index("c")
    first_plane = wid * PLANES_PER_W

    lane = lax.iota(jnp.int32, L).astype(jnp.float32)
    zeros = jnp.zeros((L,), jnp.float32)
    wmax = jnp.float32(W - 1)
    hmax = jnp.float32(H - 1)
    fzero = jnp.float32(0.0)
    fone = jnp.float32(1.0)

    def start_chunk(plane, ch, slot):
        off = plane * PLANE + ch * CHUNK
        for a, (hbm, _) in enumerate(((img_hbm, 0), (u_hbm, 1), (v_hbm, 2))):
            pltpu.async_copy(hbm.at[pl.ds(off, CHUNK)], bufs[slot][a], sems[slot])

    def wait_chunk(slot):
        for a in range(3):
            pltpu.make_async_copy(img_hbm.at[pl.ds(0, CHUNK)], bufs[slot][a],
                                  sems[slot]).wait()

    start_chunk(first_plane, 0, 0)

    def per_plane(p, carry):
        plane = first_plane + p

        for ch in range(NCHUNKS):
            slot = ch % 2
            is_last = (p == PLANES_PER_W - 1) & (ch == NCHUNKS - 1)

            @pl.when(jnp.logical_not(is_last))
            def _():
                nch = (ch + 1) % NCHUNKS
                nplane = plane + (1 if ch == NCHUNKS - 1 else 0)
                start_chunk(nplane, nch, 1 - slot)

            wait_chunk(slot)
            img_v, u_v, v_v = bufs[slot]

            def per_row(r, rcarry):
                ybase = (ch * CHUNK_ROWS + r).astype(jnp.float32)

                def load_group(g):
                    js = range(g * GROUP, min((g + 1) * GROUP, VECS_PER_ROW))
                    sls = [pl.ds(r * W + j * L, L) for j in js]
                    return ([img_v[sl] for sl in sls],
                            [u_v[sl] for sl in sls],
                            [v_v[sl] for sl in sls])

                def compute_group(g, data):
                    imgs, us, vs = data
                    js = range(g * GROUP, min((g + 1) * GROUP, VECS_PER_ROW))
                    for k, j in enumerate(js):
                        img = imgs[k]
                        X = jnp.minimum(jnp.maximum(lane + jnp.float32(j * L)
                                                    + us[k], fzero), wmax)
                        Y = jnp.minimum(jnp.maximum(ybase + vs[k], fzero), hmax)
                        x0 = X.astype(jnp.int32)
                        y0 = Y.astype(jnp.int32)
                        x0f = x0.astype(jnp.float32)
                        y0f = y0.astype(jnp.float32)
                        x1f = jnp.minimum(x0f + fone, wmax)
                        y1f = jnp.minimum(y0f + fone, hmax)
                        x1 = x1f.astype(jnp.int32)
                        y1 = y1f.astype(jnp.int32)
                        dx = X - x0f
                        dy = Y - y0f
                        gx = x1f - X
                        gy = y1f - Y
                        r0 = y0 * W
                        r1 = y1 * W
                        vdx = img * dx
                        vgx = img * gx
                        plsc.addupdate_scatter(acc, [r0 + x0], vdx * dy)
                        plsc.addupdate_scatter(acc, [r1 + x0], vdx * gy)
                        plsc.addupdate_scatter(acc, [r0 + x1], vgx * dy)
                        plsc.addupdate_scatter(acc, [r1 + x1], vgx * gy)

                ngroups = -(-VECS_PER_ROW // GROUP)
                data = load_group(0)
                for g in range(ngroups):
                    nxt = load_group(g + 1) if g + 1 < ngroups else None
                    compute_group(g, data)
                    data = nxt
                return rcarry
            lax.fori_loop(0, CHUNK_ROWS, per_row, 0)

        pltpu.sync_copy(acc, out_hbm.at[pl.ds(plane * PLANE, PLANE)])

        @pl.when(p < PLANES_PER_W - 1)
        def _rezero():
            def zero_body(i, _):
                acc[pl.ds(i * L, L)] = zeros
                return _
            lax.fori_loop(0, PLANE // L, zero_body, 0, unroll=4)
        return carry

    def zero_body0(i, _):
        acc[pl.ds(i * L, L)] = zeros
        return _
    lax.fori_loop(0, PLANE // L, zero_body0, 0, unroll=4)

    lax.fori_loop(0, PLANES_PER_W, per_plane, 0)


@jax.jit
def kernel(input_image, U, V):
    mesh = plsc.VectorSubcoreMesh(core_axis_name="c", subcore_axis_name="s",
                                  num_cores=NC, num_subcores=NS)
    run = pl.kernel(
        _advect_body,
        out_type=jax.ShapeDtypeStruct((NPLANES * PLANE,), jnp.float32),
        mesh=mesh,
        scratch_types=[
            [[pltpu.VMEM((CHUNK,), jnp.float32) for _ in range(3)]
             for _ in range(2)],
            pltpu.VMEM((PLANE,), jnp.float32),
            [pltpu.SemaphoreType.DMA for _ in range(2)],
        ],
        compiler_params=pltpu.CompilerParams(needs_layout_passes=False),
    )
    cpg = C // (NGROUPS // B)
    outs = []
    for g in range(NGROUPS):
        b, c0 = divmod(g, NGROUPS // B)
        c0 *= cpg
        outs.append(run(input_image[b, c0:c0 + cpg].reshape(-1),
                        U[b, c0:c0 + cpg].reshape(-1),
                        V[b, c0:c0 + cpg].reshape(-1)))
    return jnp.stack([o.reshape(cpg, H, W) for o in outs]).reshape(B, C, H, W)
